# baseline (device time: 158919 ns/iter reference)
import jax
import jax.numpy as jnp
from jax import lax
from jax.experimental import pallas as pl
from jax.experimental.pallas import tpu as pltpu

N_DEV = 4
M_PER = 2048
K = 1024
N = 1024
CHUNK = M_PER // N_DEV


def kernel(t, W):
    def body(t_ref, w_ref, out_ref, rs_acc, rs_recv, ag_buf, send_sems, recv_sems):
        my = lax.axis_index("i")
        left = lax.rem(my + N_DEV - 1, N_DEV)
        right = lax.rem(my + 1, N_DEV)

        barrier_sem = pltpu.get_barrier_semaphore()
        for nbr in (left, right):
            pl.semaphore_signal(
                barrier_sem, inc=1,
                device_id=(nbr,), device_id_type=pl.DeviceIdType.MESH,
            )
        pl.semaphore_wait(barrier_sem, 2)

        rs_acc[...] = t_ref[pl.ds(my * CHUNK, CHUNK), :]
        for h in range(N_DEV - 1):
            rdma = pltpu.make_async_remote_copy(
                src_ref=rs_acc,
                dst_ref=rs_recv.at[h],
                send_sem=send_sems.at[h],
                recv_sem=recv_sems.at[h],
                device_id=(right,),
                device_id_type=pl.DeviceIdType.MESH,
            )
            rdma.start()
            rdma.wait()
            nxt = lax.rem(my + N_DEV - 1 - h, N_DEV)
            rs_acc[...] = rs_recv[h] + t_ref[pl.ds(nxt * CHUNK, CHUNK), :]

        own = right

        ag_buf[0] = jnp.dot(
            rs_acc[...], w_ref[...], preferred_element_type=jnp.float32
        )
        out_ref[pl.ds(own * CHUNK, CHUNK), :] = ag_buf[0]

        for h in range(N_DEV - 1):
            rdma = pltpu.make_async_remote_copy(
                src_ref=ag_buf.at[h],
                dst_ref=ag_buf.at[h + 1],
                send_sem=send_sems.at[N_DEV - 1 + h],
                recv_sem=recv_sems.at[N_DEV - 1 + h],
                device_id=(right,),
                device_id_type=pl.DeviceIdType.MESH,
            )
            rdma.start()
            rdma.wait()
            origin = lax.rem(my + N_DEV - h, N_DEV)
            out_ref[pl.ds(origin * CHUNK, CHUNK), :] = ag_buf[h + 1]

    return pl.pallas_call(
        body,
        out_shape=jax.ShapeDtypeStruct((M_PER, N), jnp.float32),
        in_specs=[
            pl.BlockSpec(memory_space=pltpu.VMEM),
            pl.BlockSpec(memory_space=pltpu.VMEM),
        ],
        out_specs=pl.BlockSpec(memory_space=pltpu.VMEM),
        scratch_shapes=[
            pltpu.VMEM((CHUNK, K), jnp.float32),
            pltpu.VMEM((N_DEV - 1, CHUNK, K), jnp.float32),
            pltpu.VMEM((N_DEV, CHUNK, N), jnp.float32),
            pltpu.SemaphoreType.DMA((2 * (N_DEV - 1),)),
            pltpu.SemaphoreType.DMA((2 * (N_DEV - 1),)),
        ],
        compiler_params=pltpu.CompilerParams(collective_id=0),
    )(t, W)


# device time: 91715 ns/iter; 1.7327x vs baseline; 1.7327x over previous
import jax
import jax.numpy as jnp
from jax import lax
from jax.experimental import pallas as pl
from jax.experimental.pallas import tpu as pltpu

N_DEV = 4
M_PER = 2048
K = 1024
N = 1024
HALF = M_PER // 2
CHUNK = HALF // N_DEV
N_HOP = N_DEV - 1


def kernel(t, W):
    def body(
        t_ref, w_ref, out_ref,
        accT, recvT, agT,
        accB, recvB, agB,
        send_sems, recv_sems,
    ):
        my = lax.axis_index("i")
        left = lax.rem(my + N_DEV - 1, N_DEV)
        right = lax.rem(my + 1, N_DEV)

        barrier_sem = pltpu.get_barrier_semaphore()
        for nbr in (left, right):
            pl.semaphore_signal(
                barrier_sem, inc=1,
                device_id=(nbr,), device_id_type=pl.DeviceIdType.MESH,
            )
        pl.semaphore_wait(barrier_sem, 2)

        def t_chunk(ring, c):
            return t_ref[pl.ds(ring * HALF + c * CHUNK, CHUNK), :]

        accT[...] = t_chunk(0, my)
        accB[...] = t_chunk(1, my)
        for h in range(N_HOP):
            cw = pltpu.make_async_remote_copy(
                src_ref=accT, dst_ref=recvT.at[h],
                send_sem=send_sems.at[h], recv_sem=recv_sems.at[h],
                device_id=(right,), device_id_type=pl.DeviceIdType.MESH,
            )
            ccw = pltpu.make_async_remote_copy(
                src_ref=accB, dst_ref=recvB.at[h],
                send_sem=send_sems.at[N_HOP + h],
                recv_sem=recv_sems.at[N_HOP + h],
                device_id=(left,), device_id_type=pl.DeviceIdType.MESH,
            )
            cw.start()
            ccw.start()
            cw.wait()
            ccw.wait()
            nxtT = lax.rem(my + N_DEV - 1 - h, N_DEV)
            nxtB = lax.rem(my + 1 + h, N_DEV)
            accT[...] = recvT[h] + t_chunk(0, nxtT)
            accB[...] = recvB[h] + t_chunk(1, nxtB)

        ownT = right
        ownB = left

        agT[0] = jnp.dot(accT[...], w_ref[...], preferred_element_type=jnp.float32)
        agB[0] = jnp.dot(accB[...], w_ref[...], preferred_element_type=jnp.float32)
        out_ref[pl.ds(ownT * CHUNK, CHUNK), :] = agT[0]
        out_ref[pl.ds(HALF + ownB * CHUNK, CHUNK), :] = agB[0]

        for h in range(N_HOP):
            cw = pltpu.make_async_remote_copy(
                src_ref=agT.at[h], dst_ref=agT.at[h + 1],
                send_sem=send_sems.at[2 * N_HOP + h],
                recv_sem=recv_sems.at[2 * N_HOP + h],
                device_id=(right,), device_id_type=pl.DeviceIdType.MESH,
            )
            ccw = pltpu.make_async_remote_copy(
                src_ref=agB.at[h], dst_ref=agB.at[h + 1],
                send_sem=send_sems.at[3 * N_HOP + h],
                recv_sem=recv_sems.at[3 * N_HOP + h],
                device_id=(left,), device_id_type=pl.DeviceIdType.MESH,
            )
            cw.start()
            ccw.start()
            cw.wait()
            ccw.wait()
            origT = lax.rem(my + N_DEV - h, N_DEV)
            origB = lax.rem(my + h, N_DEV)
            out_ref[pl.ds(origT * CHUNK, CHUNK), :] = agT[h + 1]
            out_ref[pl.ds(HALF + origB * CHUNK, CHUNK), :] = agB[h + 1]

    return pl.pallas_call(
        body,
        out_shape=jax.ShapeDtypeStruct((M_PER, N), jnp.float32),
        in_specs=[
            pl.BlockSpec(memory_space=pltpu.VMEM),
            pl.BlockSpec(memory_space=pltpu.VMEM),
        ],
        out_specs=pl.BlockSpec(memory_space=pltpu.VMEM),
        scratch_shapes=[
            pltpu.VMEM((CHUNK, K), jnp.float32),
            pltpu.VMEM((N_HOP, CHUNK, K), jnp.float32),
            pltpu.VMEM((N_DEV, CHUNK, N), jnp.float32),
            pltpu.VMEM((CHUNK, K), jnp.float32),
            pltpu.VMEM((N_HOP, CHUNK, K), jnp.float32),
            pltpu.VMEM((N_DEV, CHUNK, N), jnp.float32),
            pltpu.SemaphoreType.DMA((4 * N_HOP,)),
            pltpu.SemaphoreType.DMA((4 * N_HOP,)),
        ],
        compiler_params=pltpu.CompilerParams(collective_id=0),
    )(t, W)


# device time: 81575 ns/iter; 1.9481x vs baseline; 1.1243x over previous
import jax
import jax.numpy as jnp
from jax import lax
from jax.experimental import pallas as pl
from jax.experimental.pallas import tpu as pltpu

N_DEV = 4
M_PER = 2048
K = 1024
N = 1024
HALF = M_PER // 2
CHUNK = HALF // N_DEV
N_HOP = N_DEV - 1
SUBS = 2
SUB = CHUNK // SUBS


def kernel(t, W):
    def body(
        t_ref, w_ref, out_ref,
        accT, recvT, agT,
        accB, recvB, agB,
        send_sems, recv_sems,
    ):
        my = lax.axis_index("i")
        left = lax.rem(my + N_DEV - 1, N_DEV)
        right = lax.rem(my + 1, N_DEV)

        barrier_sem = pltpu.get_barrier_semaphore()
        for nbr in (left, right):
            pl.semaphore_signal(
                barrier_sem, inc=1,
                device_id=(nbr,), device_id_type=pl.DeviceIdType.MESH,
            )
        pl.semaphore_wait(barrier_sem, 2)

        rings = (
            (accT, recvT, agT, 0, right),
            (accB, recvB, agB, 1, left),
        )

        def t_sub(ring_idx, c, s):
            return t_ref[pl.ds(ring_idx * HALF + c * CHUNK + s * SUB, SUB), :]

        def out_rows(ring_idx, c, s):
            return pl.ds(ring_idx * HALF + c * CHUNK + s * SUB, SUB)

        def sem_idx(phase, ring_idx, h, s):
            return ((phase * 2 + ring_idx) * N_HOP + h) * SUBS + s

        def rs_chunk(ring_idx, j):
            if ring_idx == 0:
                return lax.rem(my + N_DEV - j, N_DEV)
            return lax.rem(my + j, N_DEV)

        def make_rs(acc, recv, ring_idx, dst, h, s):
            return pltpu.make_async_remote_copy(
                src_ref=acc.at[s],
                dst_ref=recv.at[h, s],
                send_sem=send_sems.at[sem_idx(0, ring_idx, h, s)],
                recv_sem=recv_sems.at[sem_idx(0, ring_idx, h, s)],
                device_id=(dst,), device_id_type=pl.DeviceIdType.MESH,
            )

        def make_ag(ag, ring_idx, dst, h, s):
            return pltpu.make_async_remote_copy(
                src_ref=ag.at[h, s],
                dst_ref=ag.at[h + 1, s],
                send_sem=send_sems.at[sem_idx(1, ring_idx, h, s)],
                recv_sem=recv_sems.at[sem_idx(1, ring_idx, h, s)],
                device_id=(dst,), device_id_type=pl.DeviceIdType.MESH,
            )

        rs_d = {}
        ag_dd = {}

        for acc, recv, ag, r, dst in rings:
            for s in range(SUBS):
                acc[s] = t_sub(r, my, s)
                d = rs_d[(r, 0, s)] = make_rs(acc, recv, r, dst, 0, s)
                d.start()

        for h in range(N_HOP):
            for s in range(SUBS):
                for acc, recv, ag, r, dst in rings:
                    d = rs_d[(r, h, s)]
                    d.wait_recv()
                    d.wait_send()
                    nxt = rs_chunk(r, h + 1)
                    acc[s] = recv[h, s] + t_sub(r, nxt, s)
                    if h + 1 < N_HOP:
                        d2 = rs_d[(r, h + 1, s)] = make_rs(acc, recv, r, dst, h + 1, s)
                        d2.start()
                    else:
                        y = jnp.dot(
                            acc[s], w_ref[...],
                            preferred_element_type=jnp.float32,
                        )
                        ag[0, s] = y
                        out_ref[out_rows(r, nxt, s), :] = y
                        d2 = ag_dd[(r, 0, s)] = make_ag(ag, r, dst, 0, s)
                        d2.start()

        for h in range(N_HOP):
            for s in range(SUBS):
                for acc, recv, ag, r, dst in rings:
                    d = ag_dd[(r, h, s)]
                    d.wait_recv()
                    if h + 1 < N_HOP:
                        d2 = ag_dd[(r, h + 1, s)] = make_ag(ag, r, dst, h + 1, s)
                        d2.start()
                    orig = rs_chunk(r, h)
                    out_ref[out_rows(r, orig, s), :] = ag[h + 1, s]

        for h in range(N_HOP):
            for s in range(SUBS):
                for r in (0, 1):
                    ag_dd[(r, h, s)].wait_send()

    return pl.pallas_call(
        body,
        out_shape=jax.ShapeDtypeStruct((M_PER, N), jnp.float32),
        in_specs=[
            pl.BlockSpec(memory_space=pltpu.VMEM),
            pl.BlockSpec(memory_space=pltpu.VMEM),
        ],
        out_specs=pl.BlockSpec(memory_space=pltpu.VMEM),
        scratch_shapes=[
            pltpu.VMEM((SUBS, SUB, K), jnp.float32),
            pltpu.VMEM((N_HOP, SUBS, SUB, K), jnp.float32),
            pltpu.VMEM((N_DEV, SUBS, SUB, N), jnp.float32),
            pltpu.VMEM((SUBS, SUB, K), jnp.float32),
            pltpu.VMEM((N_HOP, SUBS, SUB, K), jnp.float32),
            pltpu.VMEM((N_DEV, SUBS, SUB, N), jnp.float32),
            pltpu.SemaphoreType.DMA((2 * 2 * N_HOP * SUBS,)),
            pltpu.SemaphoreType.DMA((2 * 2 * N_HOP * SUBS,)),
        ],
        compiler_params=pltpu.CompilerParams(collective_id=0),
    )(t, W)
